# Initial kernel scaffold; baseline (speedup 1.0000x reference)
import jax
import jax.numpy as jnp
from jax.experimental import pallas as pl


def kernel(x_target, x_drug, W_tt, a_src_tt, a_dst_tt, b_tt, W_dt_src, W_dt_dst, a_src_dt, a_dst_dt, b_dt, W_dd, a_src_dd, a_dst_dd, b_dd, W_td_src, W_td_dst, a_src_td, a_dst_td, b_td, edge_index_tt, edge_index_dt, edge_index_dd, edge_index_td, edge_attr_dummy):
    raise NotImplementedError("write your pallas kernel here")


# speedup vs baseline: 21.1829x; 21.1829x over previous
import jax
import jax.numpy as jnp
from jax import lax
from jax.experimental import pallas as pl
from jax.experimental.pallas import tpu as pltpu
from jax.experimental.pallas import tpu_sc as plsc

N_T = 10000
N_D = 10000
E = 320000
D = 128
H = 128
NEG_SLOPE = 0.2

NC = 2
NS = 16
LANES = 16

N_PAD = 10240
CHUNK = 128
CHUNKS_PER_TILE = 157
E_PAD = NS * CHUNKS_PER_TILE * CHUNK
HH = H // 2
QW = H // 4
EPS = 1e-16

_f32 = jnp.float32


def _proj_a_body(xt_ref, xd_ref, wtt, wdts, wdd, wdtd, wtdd,
                 pa, pb, pc, pd, pe,
                 htt_o, hsdt_o, hdd_o, at_o, ad_o):
    xt = xt_ref[...]
    xd = xd_ref[...]
    htt = jnp.dot(xt, wtt[...], preferred_element_type=_f32)
    hsdt = jnp.dot(xd, wdts[...], preferred_element_type=_f32)
    hdd = jnp.dot(xd, wdd[...], preferred_element_type=_f32)
    hddt = jnp.dot(xt, wdtd[...], preferred_element_type=_f32)
    hdtd = jnp.dot(xd, wtdd[...], preferred_element_type=_f32)
    htt_o[...] = htt
    hsdt_o[...] = hsdt
    hdd_o[...] = hdd
    at_o[...] = (jnp.dot(htt, pa[...], preferred_element_type=_f32)
                 + jnp.dot(hddt, pb[...], preferred_element_type=_f32))
    ad_o[...] = (jnp.dot(hsdt, pc[...], preferred_element_type=_f32)
                 + jnp.dot(hdd, pd[...], preferred_element_type=_f32)
                 + jnp.dot(hdtd, pe[...], preferred_element_type=_f32))


def _final_d_body(odd_ref, ytd_ref, xtn_ref, wtds, btd, out_ref, xtn_out):
    proj = jnp.dot(ytd_ref[...], wtds[...], preferred_element_type=_f32)
    out_ref[...] = (odd_ref[...] + proj + btd[...]) * 0.5
    xtn_out[...] = xtn_ref[...]


_BR = 2000


def _row_spec():
    return pl.BlockSpec((_BR, D), lambda i: (i, 0))


def _w_spec():
    return pl.BlockSpec((D, H), lambda i: (0, 0))


def _tc_proj_a(x_t, x_d, w_tt, w_dt_src, w_dd, w_dt_dst, w_td_dst,
               pa, pb, pc, pd, pe):
    out_sd = jax.ShapeDtypeStruct((N_T, H), _f32)
    return pl.pallas_call(
        _proj_a_body,
        grid=(N_T // _BR,),
        in_specs=[_row_spec(), _row_spec()] + [_w_spec()] * 10,
        out_specs=[_row_spec()] * 5,
        out_shape=[out_sd] * 5,
    )(x_t, x_d, w_tt, w_dt_src, w_dd, w_dt_dst, w_td_dst, pa, pb, pc, pd, pe)


def _tc_final_d(odd, ytd, xtn, w_td_src, b_td):
    return pl.pallas_call(
        _final_d_body,
        grid=(N_D // _BR,),
        in_specs=[_row_spec(), _row_spec(), _row_spec(), _w_spec(),
                  pl.BlockSpec((1, H), lambda i: (0, 0))],
        out_specs=[_row_spec()] * 2,
        out_shape=[jax.ShapeDtypeStruct((N_D, H), _f32),
                   jax.ShapeDtypeStruct((N_T, H), _f32)],
    )(odd, ytd, xtn, w_td_src, b_td)


def _sc_body(as_tt, ad_tt, s_tt, j_tt, i_tt, hs_tt, b_tt,
             as_dt, ad_dt, s_dt, j_dt, i_dt, hs_dt, b_dt,
             as_dd, ad_dd, j_dd, i_dd, hs_dd, b_dd,
             ad_td, j_td, i_td,
             xtn_lo, xtn_hi, odd_lo, odd_hi, ytd_lo, ytd_hi,
             asv, adv, sv, jv, iv, pbuf, qbuf, rows,
             fout, fd, qd, qd2, bs, bs2, oidx,
             acc, den, acs_a, acs_b,
             sem):
    cid = lax.axis_index("c")
    tid = lax.axis_index("s")
    zero16 = jnp.zeros((LANES,), _f32)
    n_blk = N_PAD // NS // CHUNK
    rows_per_tile = N_PAD // NS

    def _zero_acc():
        def _zrow(r, _):
            for q in range(QW // LANES):
                rows[r, pl.ds(q * LANES, LANES)] = zero16
            return _
        lax.fori_loop(0, CHUNK, _zrow, None)

        def _zblk(blk, _):
            base = tid * rows_per_tile + blk * CHUNK
            pltpu.sync_copy(rows, acc.at[pl.ds(base, CHUNK)])
            return _
        lax.fori_loop(0, n_blk, _zblk, None)

    def _zero_1d(ref):
        for q in range(CHUNK // LANES):
            pbuf[pl.ds(q * LANES, LANES)] = zero16

        def _zblk(blk, _):
            base = tid * rows_per_tile + blk * CHUNK
            pltpu.sync_copy(pbuf, ref.at[pl.ds(base, CHUNK)])
            return _
        lax.fori_loop(0, n_blk, _zblk, None)

    def _stage(as_src, ad_src, s_src, j_h, i_h):
        pltpu.sync_copy(as_src, asv)
        pltpu.sync_copy(ad_src, adv)
        if s_src is not None:
            pltpu.sync_copy(s_src, sv)
        pltpu.sync_copy(j_h.at[tid], jv)
        pltpu.sync_copy(i_h.at[tid], iv)

    def _edge_pass(hs_h, shift, qq, acs):
        def _chunk(ch, _):
            cp = pltpu.async_copy(hs_h.at[jv.at[ch]], rows, sem)
            for k in range(CHUNK // LANES):
                sl = pl.ds(k * LANES, LANES)
                jj = lax.shift_right_logical(jv[ch, sl], shift)
                ii = iv[ch, sl]
                e = (plsc.load_gather(asv, [jj])
                     + plsc.load_gather(adv, [ii]))
                e = jnp.where(e > 0, e, e * NEG_SLOPE)
                p = jnp.exp(e)
                pbuf[sl] = p
                if acs is not None:
                    qbuf[sl] = p * plsc.load_gather(sv, [jj])

            @pl.when(qq == 0)
            def _():
                pltpu.sync_copy(pbuf, den.at[iv.at[ch]], add=True)
                if acs is not None:
                    pltpu.sync_copy(qbuf, acs.at[iv.at[ch]], add=True)
            cp.wait()

            def _scale(k, _):
                p16 = pbuf[pl.ds(k * LANES, LANES)]
                for r in range(LANES):
                    ps = p16[r]
                    for q in range(QW // LANES):
                        sl = pl.ds(q * LANES, LANES)
                        rows[k * LANES + r, sl] = rows[k * LANES + r, sl] * ps
                return _
            lax.fori_loop(0, CHUNK // LANES, _scale, None)
            pltpu.sync_copy(rows, acc.at[iv.at[ch]], add=True)
            return _
        lax.fori_loop(0, CHUNKS_PER_TILE, _chunk, None)

    def _finalize(out_ref, qq, bias_buf, readback, extra):
        iota = lax.iota(jnp.int32, LANES)

        def _blk_body(blk, _):
            base = tid * rows_per_tile + blk * CHUNK
            for k in range(CHUNK // LANES):
                oidx[pl.ds(k * LANES, LANES)] = (
                    (base + k * LANES + iota) * 2 + qq)
            pltpu.sync_copy(acc.at[pl.ds(base, CHUNK)], rows)
            pltpu.sync_copy(den.at[pl.ds(base, CHUNK)], fd)
            if readback:
                pltpu.sync_copy(out_ref.at[oidx], fout)

            def _fin(k, _):
                ra16 = 1.0 / (fd[pl.ds(k * LANES, LANES)] + EPS)
                for r in range(LANES):
                    ra = ra16[r]
                    rr = k * LANES + r
                    for q in range(QW // LANES):
                        sl = pl.ds(q * LANES, LANES)
                        v = rows[rr, sl] * ra
                        if bias_buf is not None:
                            v = v + bias_buf[pl.ds(qq * QW + q * LANES,
                                                   LANES)]
                        if readback:
                            v = (v + fout[rr, sl]) * 0.5
                        fout[rr, sl] = v
                return _
            lax.fori_loop(0, CHUNK // LANES, _fin, None)
            pltpu.sync_copy(fout, out_ref.at[oidx])
            if extra is not None:
                @pl.when(qq == 0)
                def _():
                    extra(base)
            return _
        lax.fori_loop(0, n_blk, _blk_body, None)

    def _acs_div(base):
        pltpu.sync_copy(acs_a.at[pl.ds(base, CHUNK)], qd)
        for k in range(CHUNK // LANES):
            sl = pl.ds(k * LANES, LANES)
            qd[sl] = qd[sl] * (1.0 / (fd[sl] + EPS))
        pltpu.sync_copy(qd, acs_a.at[pl.ds(base, CHUNK)])

    def _astd_mk(base):
        pltpu.sync_copy(acs_a.at[pl.ds(base, CHUNK)], qd2)
        pltpu.sync_copy(acs_b.at[pl.ds(base, CHUNK)], qd)
        for k in range(CHUNK // LANES):
            sl = pl.ds(k * LANES, LANES)
            qd[sl] = (qd2[sl] + qd[sl] * (1.0 / (fd[sl] + EPS))) * 0.5
        pltpu.sync_copy(qd, acs_a.at[pl.ds(base, CHUNK)])

    def _conv(out_lo_ref, out_hi_ref, bias_buf, readback, acs, extra,
              hs_pair, half_mode):
        mult = 2 if half_mode else 4
        shift = 1 if half_mode else 2

        def _q_iter(qq, _):
            off = qq if half_mode else 2 * cid + qq

            def _xf(r, _):
                for k in range(CHUNK // LANES):
                    sl = pl.ds(k * LANES, LANES)
                    v = jv[r, sl]
                    jv[r, sl] = jnp.where(qq == 0, v * mult + off, v + 1)
                return _
            lax.fori_loop(0, CHUNKS_PER_TILE, _xf, None)

            if half_mode:
                @pl.when(cid == 0)
                def _():
                    _edge_pass(hs_pair[0], shift, qq, acs)

                @pl.when(cid == 1)
                def _():
                    _edge_pass(hs_pair[1], shift, qq, acs)
            else:
                _edge_pass(hs_pair[0], shift, qq, acs)
            plsc.subcore_barrier()

            @pl.when(cid == 0)
            def _():
                _finalize(out_lo_ref, qq, bias_buf, readback, extra)

            @pl.when(cid == 1)
            def _():
                _finalize(out_hi_ref, qq, bias_buf, readback, extra)
            _zero_acc()
            plsc.subcore_barrier()
            return _
        lax.fori_loop(0, 2, _q_iter, None)

    pltpu.sync_copy(b_tt.at[pl.ds(cid * HH, HH)], bs)
    pltpu.sync_copy(b_dt.at[pl.ds(cid * HH, HH)], bs2)
    _zero_acc()
    _zero_1d(den)
    _zero_1d(acs_a)
    _zero_1d(acs_b)
    plsc.subcore_barrier()

    _stage(as_tt, ad_tt, s_tt, j_tt, i_tt)
    _conv(xtn_lo, xtn_hi, bs, False, acs_a, _acs_div, (hs_tt,), False)

    _zero_1d(den)
    plsc.subcore_barrier()
    _stage(as_dt, ad_dt, s_dt, j_dt, i_dt)
    _conv(xtn_lo, xtn_hi, bs2, True, acs_b, _astd_mk, (hs_dt,), False)

    _zero_1d(den)
    pltpu.sync_copy(b_dd.at[pl.ds(cid * HH, HH)], bs)
    plsc.subcore_barrier()
    _stage(as_dd, ad_dd, None, j_dd, i_dd)
    _conv(odd_lo, odd_hi, bs, False, None, None, (hs_dd,), False)

    _zero_1d(den)
    plsc.subcore_barrier()
    _stage(acs_a, ad_td, None, j_td, i_td)
    _conv(ytd_lo, ytd_hi, None, False, None, None, (xtn_lo, xtn_hi), True)


def _sc_all(as_tt, ad_tt, s_tt, j_tt, i_tt, hs_tt, b_tt,
            as_dt, ad_dt, s_dt, j_dt, i_dt, hs_dt, b_dt,
            as_dd, ad_dd, j_dd, i_dd, hs_dd, b_dd,
            ad_td, j_td, i_td):
    mesh = plsc.VectorSubcoreMesh(core_axis_name="c", subcore_axis_name="s",
                                  num_cores=NC, num_subcores=NS)
    quarter = pltpu.HBM((2 * N_PAD, QW), _f32)
    f = pl.kernel(
        _sc_body,
        out_type=[quarter] * 6,
        mesh=mesh,
        compiler_params=pltpu.CompilerParams(needs_layout_passes=False,
                                             use_tc_tiling_on_sc=False),
        scratch_types=[
            pltpu.VMEM((N_PAD,), _f32),
            pltpu.VMEM((N_PAD,), _f32),
            pltpu.VMEM((N_PAD,), _f32),
            pltpu.VMEM((CHUNKS_PER_TILE, CHUNK), jnp.int32),
            pltpu.VMEM((CHUNKS_PER_TILE, CHUNK), jnp.int32),
            pltpu.VMEM((CHUNK,), _f32),
            pltpu.VMEM((CHUNK,), _f32),
            pltpu.VMEM((CHUNK, QW), _f32),
            pltpu.VMEM((CHUNK, QW), _f32),
            pltpu.VMEM((CHUNK,), _f32),
            pltpu.VMEM((CHUNK,), _f32),
            pltpu.VMEM((CHUNK,), _f32),
            pltpu.VMEM((HH,), _f32),
            pltpu.VMEM((HH,), _f32),
            pltpu.VMEM((CHUNK,), jnp.int32),
            pltpu.VMEM_SHARED((N_PAD, QW), _f32),
            pltpu.VMEM_SHARED((N_PAD,), _f32),
            pltpu.VMEM_SHARED((N_PAD,), _f32),
            pltpu.VMEM_SHARED((N_PAD,), _f32),
            pltpu.SemaphoreType.DMA,
        ],
    )
    return f(as_tt, ad_tt, s_tt, j_tt, i_tt, hs_tt, b_tt,
             as_dt, ad_dt, s_dt, j_dt, i_dt, hs_dt, b_dt,
             as_dd, ad_dd, j_dd, i_dd, hs_dd, b_dd,
             ad_td, j_td, i_td)


def _pad_alpha(v):
    return jnp.pad(v, (0, N_PAD - v.shape[0]))


def _prep_edges(ei, garbage_dst):
    pad = E_PAD - E
    j = jnp.concatenate([ei[0], jnp.zeros((pad,), jnp.int32)])
    i = jnp.concatenate([ei[1], jnp.full((pad,), garbage_dst, jnp.int32)])
    return (j.reshape(NS, CHUNKS_PER_TILE, CHUNK),
            i.reshape(NS, CHUNKS_PER_TILE, CHUNK))


def _col_pack(*cols):
    p = jnp.zeros((H, H), _f32)
    for idx, c in cols:
        p = p.at[:, idx].set(c)
    return p


def _unquarter(a, n):
    return a.reshape(N_PAD, HH)[:n]


def kernel(x_target, x_drug, W_tt, a_src_tt, a_dst_tt, b_tt,
           W_dt_src, W_dt_dst, a_src_dt, a_dst_dt, b_dt,
           W_dd, a_src_dd, a_dst_dd, b_dd,
           W_td_src, W_td_dst, a_src_td, a_dst_td, b_td,
           edge_index_tt, edge_index_dt, edge_index_dd, edge_index_td,
           edge_attr_dummy):
    w_td = W_td_src @ a_src_td
    gamma = 0.5 * jnp.dot(b_tt + b_dt, w_td)

    pa = _col_pack((0, a_src_tt), (1, a_dst_tt), (3, w_td))
    pb = _col_pack((2, a_dst_dt))
    pc = _col_pack((0, a_src_dt), (4, w_td))
    pd = _col_pack((1, a_src_dd), (2, a_dst_dd))
    pe = _col_pack((3, a_dst_td))

    htt, hsdt, hdd, alpha_t, alpha_d = _tc_proj_a(
        x_target, x_drug, W_tt, W_dt_src, W_dd, W_dt_dst, W_td_dst,
        pa, pb, pc, pd, pe)

    j_tt, i_tt = _prep_edges(edge_index_tt, N_T)
    j_dt, i_dt = _prep_edges(edge_index_dt, N_T)
    j_dd, i_dd = _prep_edges(edge_index_dd, N_D)
    j_td, i_td = _prep_edges(edge_index_td, N_D)

    (xtn_lo, xtn_hi, odd_lo, odd_hi, ytd_lo, ytd_hi) = _sc_all(
        _pad_alpha(alpha_t[:, 0]), _pad_alpha(alpha_t[:, 1]),
        _pad_alpha(alpha_t[:, 3]), j_tt, i_tt,
        htt.reshape(4 * N_T, QW), b_tt,
        _pad_alpha(alpha_d[:, 0]), _pad_alpha(alpha_t[:, 2]),
        _pad_alpha(alpha_d[:, 4]), j_dt, i_dt,
        hsdt.reshape(4 * N_D, QW), b_dt,
        _pad_alpha(alpha_d[:, 1]), _pad_alpha(alpha_d[:, 2]),
        j_dd, i_dd, hdd.reshape(4 * N_D, QW), b_dd,
        _pad_alpha(alpha_d[:, 3] + gamma), j_td, i_td)

    x_target_new = jnp.concatenate(
        [_unquarter(xtn_lo, N_T), _unquarter(xtn_hi, N_T)], axis=1)
    odd = jnp.concatenate(
        [_unquarter(odd_lo, N_D), _unquarter(odd_hi, N_D)], axis=1)
    ytd = jnp.concatenate(
        [_unquarter(ytd_lo, N_D), _unquarter(ytd_hi, N_D)], axis=1)

    x_drug_new, x_target_new = _tc_final_d(
        odd, ytd, x_target_new, W_td_src, b_td.reshape(1, H))
    return (x_target_new, x_drug_new)


# speedup vs baseline: 31.7392x; 1.4983x over previous
import jax
import jax.numpy as jnp
from jax import lax
from jax.experimental import pallas as pl
from jax.experimental.pallas import tpu as pltpu
from jax.experimental.pallas import tpu_sc as plsc

N_T = 10000
N_D = 10000
E = 320000
D = 128
H = 128
NEG_SLOPE = 0.2

NC = 2
NS = 16
LANES = 16

N_PAD = 10240
CHUNK = 128
CHUNKS_PER_TILE = 157
E_PAD = NS * CHUNKS_PER_TILE * CHUNK
HH = H // 2
QW = H // 4
EPS = 1e-16

_f32 = jnp.float32


def _proj_a_body(xt_ref, xd_ref, wtt, wdts, wdd, wdtd, wtdd,
                 pa, pb, pc, pd, pe,
                 htt_o, hsdt_o, hdd_o, at_o, ad_o):
    xt = xt_ref[...]
    xd = xd_ref[...]
    htt = jnp.dot(xt, wtt[...], preferred_element_type=_f32)
    hsdt = jnp.dot(xd, wdts[...], preferred_element_type=_f32)
    hdd = jnp.dot(xd, wdd[...], preferred_element_type=_f32)
    hddt = jnp.dot(xt, wdtd[...], preferred_element_type=_f32)
    hdtd = jnp.dot(xd, wtdd[...], preferred_element_type=_f32)
    htt_o[...] = htt
    hsdt_o[...] = hsdt
    hdd_o[...] = hdd
    at_o[...] = (jnp.dot(htt, pa[...], preferred_element_type=_f32)
                 + jnp.dot(hddt, pb[...], preferred_element_type=_f32))
    ad_o[...] = (jnp.dot(hsdt, pc[...], preferred_element_type=_f32)
                 + jnp.dot(hdd, pd[...], preferred_element_type=_f32)
                 + jnp.dot(hdtd, pe[...], preferred_element_type=_f32))


def _final_d_body(odd_ref, ytd_ref, xtn_ref, wtds, btd, out_ref, xtn_out):
    proj = jnp.dot(ytd_ref[...], wtds[...], preferred_element_type=_f32)
    out_ref[...] = (odd_ref[...] + proj + btd[...]) * 0.5
    xtn_out[...] = xtn_ref[...]


_BR = 2000


def _row_spec():
    return pl.BlockSpec((_BR, D), lambda i: (i, 0))


def _w_spec():
    return pl.BlockSpec((D, H), lambda i: (0, 0))


def _tc_proj_a(x_t, x_d, w_tt, w_dt_src, w_dd, w_dt_dst, w_td_dst,
               pa, pb, pc, pd, pe):
    out_sd = jax.ShapeDtypeStruct((N_T, H), _f32)
    return pl.pallas_call(
        _proj_a_body,
        grid=(N_T // _BR,),
        in_specs=[_row_spec(), _row_spec()] + [_w_spec()] * 10,
        out_specs=[_row_spec()] * 5,
        out_shape=[out_sd] * 5,
    )(x_t, x_d, w_tt, w_dt_src, w_dd, w_dt_dst, w_td_dst, pa, pb, pc, pd, pe)


def _tc_final_d(odd, ytd, xtn, w_td_src, b_td):
    return pl.pallas_call(
        _final_d_body,
        grid=(N_D // _BR,),
        in_specs=[_row_spec(), _row_spec(), _row_spec(), _w_spec(),
                  pl.BlockSpec((1, H), lambda i: (0, 0))],
        out_specs=[_row_spec()] * 2,
        out_shape=[jax.ShapeDtypeStruct((N_D, H), _f32),
                   jax.ShapeDtypeStruct((N_T, H), _f32)],
    )(odd, ytd, xtn, w_td_src, b_td)


def _sc_body(as_tt, ad_tt, s_tt, j_tt, i_tt, hs_tt, b_tt,
             as_dt, ad_dt, s_dt, j_dt, i_dt, hs_dt, b_dt,
             as_dd, ad_dd, j_dd, i_dd, hs_dd, b_dd,
             ad_td, j_td, i_td,
             xtn_lo, xtn_hi, odd_lo, odd_hi, ytd_lo, ytd_hi,
             asv, adv, sv, jv, iv, pbuf, qbuf, pbuf2, qbuf2, rows, rows2,
             fout, fd, qd, qd2, bs, bs2, oidx,
             acc, den, acs_a, acs_b,
             sem, sem2, sem_s, sem_s2, sem_d, sem_d2, sem_e, sem_e2):
    cid = lax.axis_index("c")
    tid = lax.axis_index("s")
    zero16 = jnp.zeros((LANES,), _f32)
    n_blk = N_PAD // NS // CHUNK
    rows_per_tile = N_PAD // NS

    def _zero_acc():
        def _zrow(r, _):
            for q in range(QW // LANES):
                rows[r, pl.ds(q * LANES, LANES)] = zero16
            return _
        lax.fori_loop(0, CHUNK, _zrow, None)

        def _zblk(blk, _):
            base = tid * rows_per_tile + blk * CHUNK
            pltpu.sync_copy(rows, acc.at[pl.ds(base, CHUNK)])
            return _
        lax.fori_loop(0, n_blk, _zblk, None)

    def _zero_1d(ref):
        for q in range(CHUNK // LANES):
            pbuf[pl.ds(q * LANES, LANES)] = zero16

        def _zblk(blk, _):
            base = tid * rows_per_tile + blk * CHUNK
            pltpu.sync_copy(pbuf, ref.at[pl.ds(base, CHUNK)])
            return _
        lax.fori_loop(0, n_blk, _zblk, None)

    def _stage(as_src, ad_src, s_src, j_h, i_h):
        pltpu.sync_copy(as_src, asv)
        pltpu.sync_copy(ad_src, adv)
        if s_src is not None:
            pltpu.sync_copy(s_src, sv)
        pltpu.sync_copy(j_h.at[tid], jv)
        pltpu.sync_copy(i_h.at[tid], iv)

    def _edge_pass(hs_h, shift, qq, acs):
        def _logits(ch, pb, qb, dsem, esem):
            @pl.when(jnp.logical_and(qq == 0, ch >= 2))
            def _():
                pltpu.make_async_copy(pb, den.at[iv.at[ch]], dsem).wait()
                if acs is not None:
                    pltpu.make_async_copy(qb, acs.at[iv.at[ch]], esem).wait()
            for k in range(CHUNK // LANES):
                sl = pl.ds(k * LANES, LANES)
                jj = lax.shift_right_logical(jv[ch, sl], shift)
                ii = iv[ch, sl]
                e = (plsc.load_gather(asv, [jj])
                     + plsc.load_gather(adv, [ii]))
                e = jnp.where(e > 0, e, e * NEG_SLOPE)
                p = jnp.exp(e)
                pb[sl] = p
                if acs is not None:
                    qb[sl] = p * plsc.load_gather(sv, [jj])

            @pl.when(qq == 0)
            def _():
                pltpu.async_copy(pb, den.at[iv.at[ch]], dsem, add=True)
                if acs is not None:
                    pltpu.async_copy(qb, acs.at[iv.at[ch]], esem, add=True)

        def _scale(ch, buf, pb):
            def _sc(k, _):
                p16 = pb[pl.ds(k * LANES, LANES)]
                for r in range(LANES):
                    ps = p16[r]
                    for q in range(QW // LANES):
                        sl = pl.ds(q * LANES, LANES)
                        buf[k * LANES + r, sl] = buf[k * LANES + r, sl] * ps
                return _
            lax.fori_loop(0, CHUNK // LANES, _sc, None)

        pltpu.async_copy(hs_h.at[jv.at[0]], rows, sem)

        def _pair(p2, _):
            ch0 = p2 * 2
            ch1 = ch0 + 1

            @pl.when(ch1 < CHUNKS_PER_TILE)
            def _():
                @pl.when(p2 >= 1)
                def _():
                    pltpu.make_async_copy(rows2, acc.at[iv.at[ch0]],
                                          sem_s2).wait()
                pltpu.async_copy(hs_h.at[jv.at[ch1]], rows2, sem2)
            _logits(ch0, pbuf, qbuf, sem_d, sem_e)
            pltpu.make_async_copy(hs_h.at[jv.at[ch0]], rows, sem).wait()
            _scale(ch0, rows, pbuf)
            pltpu.async_copy(rows, acc.at[iv.at[ch0]], sem_s, add=True)

            @pl.when(ch1 < CHUNKS_PER_TILE)
            def _():
                _logits(ch1, pbuf2, qbuf2, sem_d2, sem_e2)
                pltpu.make_async_copy(rows, acc.at[iv.at[ch0]], sem_s).wait()

                @pl.when(ch0 + 2 < CHUNKS_PER_TILE)
                def _():
                    pltpu.async_copy(hs_h.at[jv.at[ch0 + 2]], rows, sem)
                pltpu.make_async_copy(hs_h.at[jv.at[ch1]], rows2, sem2).wait()
                _scale(ch1, rows2, pbuf2)
                pltpu.async_copy(rows2, acc.at[iv.at[ch1]], sem_s2, add=True)
            return _
        lax.fori_loop(0, (CHUNKS_PER_TILE + 1) // 2, _pair, None)
        pltpu.make_async_copy(rows, acc.at[iv.at[0]], sem_s).wait()
        pltpu.make_async_copy(rows2, acc.at[iv.at[0]], sem_s2).wait()

        @pl.when(qq == 0)
        def _():
            pltpu.make_async_copy(pbuf, den.at[iv.at[0]], sem_d).wait()
            pltpu.make_async_copy(pbuf2, den.at[iv.at[0]], sem_d2).wait()
            if acs is not None:
                pltpu.make_async_copy(qbuf, acs.at[iv.at[0]], sem_e).wait()
                pltpu.make_async_copy(qbuf2, acs.at[iv.at[0]], sem_e2).wait()

    def _finalize(out_ref, qq, bias_buf, readback, extra):
        iota = lax.iota(jnp.int32, LANES)

        def _blk_body(blk, _):
            base = tid * rows_per_tile + blk * CHUNK
            for k in range(CHUNK // LANES):
                oidx[pl.ds(k * LANES, LANES)] = (
                    (base + k * LANES + iota) * 2 + qq)
            pltpu.sync_copy(acc.at[pl.ds(base, CHUNK)], rows)
            pltpu.sync_copy(den.at[pl.ds(base, CHUNK)], fd)
            if readback:
                pltpu.sync_copy(out_ref.at[oidx], fout)

            def _fin(k, _):
                ra16 = 1.0 / (fd[pl.ds(k * LANES, LANES)] + EPS)
                for r in range(LANES):
                    ra = ra16[r]
                    rr = k * LANES + r
                    for q in range(QW // LANES):
                        sl = pl.ds(q * LANES, LANES)
                        v = rows[rr, sl] * ra
                        if bias_buf is not None:
                            v = v + bias_buf[pl.ds(qq * QW + q * LANES,
                                                   LANES)]
                        if readback:
                            v = (v + fout[rr, sl]) * 0.5
                        fout[rr, sl] = v
                return _
            lax.fori_loop(0, CHUNK // LANES, _fin, None)
            pltpu.sync_copy(fout, out_ref.at[oidx])
            if extra is not None:
                @pl.when(qq == 0)
                def _():
                    extra(base)
            return _
        lax.fori_loop(0, n_blk, _blk_body, None)

    def _acs_div(base):
        pltpu.sync_copy(acs_a.at[pl.ds(base, CHUNK)], qd)
        for k in range(CHUNK // LANES):
            sl = pl.ds(k * LANES, LANES)
            qd[sl] = qd[sl] * (1.0 / (fd[sl] + EPS))
        pltpu.sync_copy(qd, acs_a.at[pl.ds(base, CHUNK)])

    def _astd_mk(base):
        pltpu.sync_copy(acs_a.at[pl.ds(base, CHUNK)], qd2)
        pltpu.sync_copy(acs_b.at[pl.ds(base, CHUNK)], qd)
        for k in range(CHUNK // LANES):
            sl = pl.ds(k * LANES, LANES)
            qd[sl] = (qd2[sl] + qd[sl] * (1.0 / (fd[sl] + EPS))) * 0.5
        pltpu.sync_copy(qd, acs_a.at[pl.ds(base, CHUNK)])

    def _conv(out_lo_ref, out_hi_ref, bias_buf, readback, acs, extra,
              hs_pair, half_mode):
        mult = 2 if half_mode else 4
        shift = 1 if half_mode else 2

        def _q_iter(qq, _):
            off = qq if half_mode else 2 * cid + qq

            def _xf(r, _):
                for k in range(CHUNK // LANES):
                    sl = pl.ds(k * LANES, LANES)
                    v = jv[r, sl]
                    jv[r, sl] = jnp.where(qq == 0, v * mult + off, v + 1)
                return _
            lax.fori_loop(0, CHUNKS_PER_TILE, _xf, None)

            if half_mode:
                @pl.when(cid == 0)
                def _():
                    _edge_pass(hs_pair[0], shift, qq, acs)

                @pl.when(cid == 1)
                def _():
                    _edge_pass(hs_pair[1], shift, qq, acs)
            else:
                _edge_pass(hs_pair[0], shift, qq, acs)
            plsc.subcore_barrier()

            @pl.when(cid == 0)
            def _():
                _finalize(out_lo_ref, qq, bias_buf, readback, extra)

            @pl.when(cid == 1)
            def _():
                _finalize(out_hi_ref, qq, bias_buf, readback, extra)
            _zero_acc()
            plsc.subcore_barrier()
            return _
        lax.fori_loop(0, 2, _q_iter, None)

    pltpu.sync_copy(b_tt.at[pl.ds(cid * HH, HH)], bs)
    pltpu.sync_copy(b_dt.at[pl.ds(cid * HH, HH)], bs2)
    _zero_acc()
    _zero_1d(den)
    _zero_1d(acs_a)
    _zero_1d(acs_b)
    plsc.subcore_barrier()

    _stage(as_tt, ad_tt, s_tt, j_tt, i_tt)
    _conv(xtn_lo, xtn_hi, bs, False, acs_a, _acs_div, (hs_tt,), False)

    _zero_1d(den)
    plsc.subcore_barrier()
    _stage(as_dt, ad_dt, s_dt, j_dt, i_dt)
    _conv(xtn_lo, xtn_hi, bs2, True, acs_b, _astd_mk, (hs_dt,), False)

    _zero_1d(den)
    pltpu.sync_copy(b_dd.at[pl.ds(cid * HH, HH)], bs)
    plsc.subcore_barrier()
    _stage(as_dd, ad_dd, None, j_dd, i_dd)
    _conv(odd_lo, odd_hi, bs, False, None, None, (hs_dd,), False)

    _zero_1d(den)
    plsc.subcore_barrier()
    _stage(acs_a, ad_td, None, j_td, i_td)
    _conv(ytd_lo, ytd_hi, None, False, None, None, (xtn_lo, xtn_hi), True)


def _sc_all(as_tt, ad_tt, s_tt, j_tt, i_tt, hs_tt, b_tt,
            as_dt, ad_dt, s_dt, j_dt, i_dt, hs_dt, b_dt,
            as_dd, ad_dd, j_dd, i_dd, hs_dd, b_dd,
            ad_td, j_td, i_td):
    mesh = plsc.VectorSubcoreMesh(core_axis_name="c", subcore_axis_name="s",
                                  num_cores=NC, num_subcores=NS)
    quarter = pltpu.HBM((2 * N_PAD, QW), _f32)
    f = pl.kernel(
        _sc_body,
        out_type=[quarter] * 6,
        mesh=mesh,
        compiler_params=pltpu.CompilerParams(needs_layout_passes=False,
                                             use_tc_tiling_on_sc=False),
        scratch_types=[
            pltpu.VMEM((N_PAD,), _f32),
            pltpu.VMEM((N_PAD,), _f32),
            pltpu.VMEM((N_PAD,), _f32),
            pltpu.VMEM((CHUNKS_PER_TILE, CHUNK), jnp.int32),
            pltpu.VMEM((CHUNKS_PER_TILE, CHUNK), jnp.int32),
            pltpu.VMEM((CHUNK,), _f32),
            pltpu.VMEM((CHUNK,), _f32),
            pltpu.VMEM((CHUNK,), _f32),
            pltpu.VMEM((CHUNK,), _f32),
            pltpu.VMEM((CHUNK, QW), _f32),
            pltpu.VMEM((CHUNK, QW), _f32),
            pltpu.VMEM((CHUNK, QW), _f32),
            pltpu.VMEM((CHUNK,), _f32),
            pltpu.VMEM((CHUNK,), _f32),
            pltpu.VMEM((CHUNK,), _f32),
            pltpu.VMEM((HH,), _f32),
            pltpu.VMEM((HH,), _f32),
            pltpu.VMEM((CHUNK,), jnp.int32),
            pltpu.VMEM_SHARED((N_PAD, QW), _f32),
            pltpu.VMEM_SHARED((N_PAD,), _f32),
            pltpu.VMEM_SHARED((N_PAD,), _f32),
            pltpu.VMEM_SHARED((N_PAD,), _f32),
            pltpu.SemaphoreType.DMA,
            pltpu.SemaphoreType.DMA,
            pltpu.SemaphoreType.DMA,
            pltpu.SemaphoreType.DMA,
            pltpu.SemaphoreType.DMA,
            pltpu.SemaphoreType.DMA,
            pltpu.SemaphoreType.DMA,
            pltpu.SemaphoreType.DMA,
        ],
    )
    return f(as_tt, ad_tt, s_tt, j_tt, i_tt, hs_tt, b_tt,
             as_dt, ad_dt, s_dt, j_dt, i_dt, hs_dt, b_dt,
             as_dd, ad_dd, j_dd, i_dd, hs_dd, b_dd,
             ad_td, j_td, i_td)


def _pad_alpha(v):
    return jnp.pad(v, (0, N_PAD - v.shape[0]))


def _prep_edges(ei, garbage_dst):
    pad = E_PAD - E
    j = jnp.concatenate([ei[0], jnp.zeros((pad,), jnp.int32)])
    i = jnp.concatenate([ei[1], jnp.full((pad,), garbage_dst, jnp.int32)])
    return (j.reshape(NS, CHUNKS_PER_TILE, CHUNK),
            i.reshape(NS, CHUNKS_PER_TILE, CHUNK))


def _col_pack(*cols):
    p = jnp.zeros((H, H), _f32)
    for idx, c in cols:
        p = p.at[:, idx].set(c)
    return p


def _unquarter(a, n):
    return a.reshape(N_PAD, HH)[:n]


def kernel(x_target, x_drug, W_tt, a_src_tt, a_dst_tt, b_tt,
           W_dt_src, W_dt_dst, a_src_dt, a_dst_dt, b_dt,
           W_dd, a_src_dd, a_dst_dd, b_dd,
           W_td_src, W_td_dst, a_src_td, a_dst_td, b_td,
           edge_index_tt, edge_index_dt, edge_index_dd, edge_index_td,
           edge_attr_dummy):
    w_td = W_td_src @ a_src_td
    gamma = 0.5 * jnp.dot(b_tt + b_dt, w_td)

    pa = _col_pack((0, a_src_tt), (1, a_dst_tt), (3, w_td))
    pb = _col_pack((2, a_dst_dt))
    pc = _col_pack((0, a_src_dt), (4, w_td))
    pd = _col_pack((1, a_src_dd), (2, a_dst_dd))
    pe = _col_pack((3, a_dst_td))

    htt, hsdt, hdd, alpha_t, alpha_d = _tc_proj_a(
        x_target, x_drug, W_tt, W_dt_src, W_dd, W_dt_dst, W_td_dst,
        pa, pb, pc, pd, pe)

    j_tt, i_tt = _prep_edges(edge_index_tt, N_T)
    j_dt, i_dt = _prep_edges(edge_index_dt, N_T)
    j_dd, i_dd = _prep_edges(edge_index_dd, N_D)
    j_td, i_td = _prep_edges(edge_index_td, N_D)

    (xtn_lo, xtn_hi, odd_lo, odd_hi, ytd_lo, ytd_hi) = _sc_all(
        _pad_alpha(alpha_t[:, 0]), _pad_alpha(alpha_t[:, 1]),
        _pad_alpha(alpha_t[:, 3]), j_tt, i_tt,
        htt.reshape(4 * N_T, QW), b_tt,
        _pad_alpha(alpha_d[:, 0]), _pad_alpha(alpha_t[:, 2]),
        _pad_alpha(alpha_d[:, 4]), j_dt, i_dt,
        hsdt.reshape(4 * N_D, QW), b_dt,
        _pad_alpha(alpha_d[:, 1]), _pad_alpha(alpha_d[:, 2]),
        j_dd, i_dd, hdd.reshape(4 * N_D, QW), b_dd,
        _pad_alpha(alpha_d[:, 3] + gamma), j_td, i_td)

    x_target_new = jnp.concatenate(
        [_unquarter(xtn_lo, N_T), _unquarter(xtn_hi, N_T)], axis=1)
    odd = jnp.concatenate(
        [_unquarter(odd_lo, N_D), _unquarter(odd_hi, N_D)], axis=1)
    ytd = jnp.concatenate(
        [_unquarter(ytd_lo, N_D), _unquarter(ytd_hi, N_D)], axis=1)

    x_drug_new, x_target_new = _tc_final_d(
        odd, ytd, x_target_new, W_td_src, b_td.reshape(1, H))
    return (x_target_new, x_drug_new)
